# baseline (speedup 1.0000x reference)
import functools

import jax
import jax.numpy as jnp
from jax import lax
from jax.experimental import pallas as pl
from jax.experimental.pallas import tpu as pltpu
from jax.experimental.pallas import tpu_sc as plsc


def _transform_body(t_ref, w_ref, b_ref, o_ref):
    o_ref[...] = lax.dot_general(
        t_ref[...], w_ref[...],
        (((1,), (1,)), ((), ())),
        preferred_element_type=jnp.float32,
    ) + b_ref[...]


def _transform_table(table, W, b2):
    V, H = table.shape
    R = 10000
    assert V % R == 0
    return pl.pallas_call(
        _transform_body,
        grid=(V // R,),
        in_specs=[
            pl.BlockSpec((R, H), lambda i: (i, 0)),
            pl.BlockSpec((H, H), lambda i: (0, 0)),
            pl.BlockSpec((1, H), lambda i: (0, 0)),
        ],
        out_specs=pl.BlockSpec((R, H), lambda i: (i, 0)),
        out_shape=jax.ShapeDtypeStruct((V, H), jnp.float32),
    )(table, W, b2)


_NC = 2
_NS = 16
_NW = _NC * _NS
_CHUNK = 128


def _make_gather(N, H):
    per_w = N // _NW
    nch = per_w // _CHUNK
    assert per_w * _NW == N and nch * _CHUNK == per_w

    mesh = plsc.VectorSubcoreMesh(core_axis_name="c", subcore_axis_name="s")

    @functools.partial(
        pl.kernel,
        mesh=mesh,
        out_type=jax.ShapeDtypeStruct((N, H), jnp.float32),
        compiler_params=pltpu.CompilerParams(use_tc_tiling_on_sc=False),
        scratch_types=[
            pltpu.VMEM((per_w,), jnp.int32),
            pltpu.VMEM((_CHUNK, H), jnp.float32),
            pltpu.SemaphoreType.DMA,
        ],
    )
    def gather_k(t2_hbm, idx_hbm, out_hbm, idx_v, rows_v, sem):
        wid = lax.axis_index("s") * _NC + lax.axis_index("c")
        base = wid * per_w
        pltpu.sync_copy(idx_hbm.at[pl.ds(base, per_w)], idx_v)

        def body(g, carry):
            off = g * _CHUNK
            idx_row = idx_v.at[pl.ds(off, _CHUNK)]
            pltpu.async_copy(t2_hbm.at[idx_row], rows_v, sem).wait()
            pltpu.sync_copy(rows_v, out_hbm.at[pl.ds(base + off, _CHUNK)])
            return carry

        lax.fori_loop(0, nch, body, 0)

    return gather_k


def kernel(seq, table, W, b):
    B, L = seq.shape
    V, H = table.shape
    t2 = _transform_table(table, W, b.reshape(1, H))
    flat_idx = seq.reshape(B * L)
    out = _make_gather(B * L, H)(t2, flat_idx)
    return out.reshape(B, L, H)


# speedup vs baseline: 1.7422x; 1.7422x over previous
import functools

import jax
import jax.numpy as jnp
from jax import lax
from jax.experimental import pallas as pl
from jax.experimental.pallas import tpu as pltpu
from jax.experimental.pallas import tpu_sc as plsc


_C1 = 8192


def _transform_body(t_ref, w_ref, b_ref, o_ref):
    y = lax.dot_general(
        t_ref[...], w_ref[...],
        (((0,), (1,)), ((), ())),
        preferred_element_type=jnp.float32,
    ) + b_ref[...]
    o_ref[:, 0:64] = y
    o_ref[:, 64:128] = y


def _transform_table(table_t, W, b2):
    H, V = table_t.shape
    grid = (V + _C1 - 1) // _C1
    return pl.pallas_call(
        _transform_body,
        grid=(grid,),
        in_specs=[
            pl.BlockSpec((H, _C1), lambda i: (0, i)),
            pl.BlockSpec((H, H), lambda i: (0, 0)),
            pl.BlockSpec((1, H), lambda i: (0, 0)),
        ],
        out_specs=pl.BlockSpec((_C1, 2 * H), lambda i: (i, 0)),
        out_shape=jax.ShapeDtypeStruct((V, 2 * H), jnp.float32),
    )(table_t, W, b2)


_NC = 2
_NS = 16
_NW = _NC * _NS
_CHUNK = 128


def _make_gather(N, W2):
    per_w = N // _NW
    nch = per_w // _CHUNK
    assert per_w * _NW == N and nch * _CHUNK == per_w

    mesh = plsc.VectorSubcoreMesh(core_axis_name="c", subcore_axis_name="s")

    @functools.partial(
        pl.kernel,
        mesh=mesh,
        out_type=jax.ShapeDtypeStruct((N, W2), jnp.float32),
        scratch_types=[
            pltpu.VMEM((per_w,), jnp.int32),
            pltpu.VMEM((_CHUNK, W2), jnp.float32),
            pltpu.SemaphoreType.DMA,
        ],
    )
    def gather_k(t2_hbm, idx_hbm, out_hbm, idx_v, rows_v, sem):
        wid = lax.axis_index("s") * _NC + lax.axis_index("c")
        base = wid * per_w
        pltpu.sync_copy(idx_hbm.at[pl.ds(base, per_w)], idx_v)

        def body(g, carry):
            off = g * _CHUNK
            pltpu.async_copy(
                t2_hbm.at[idx_v.at[pl.ds(off, _CHUNK)]], rows_v, sem).wait()
            pltpu.sync_copy(rows_v, out_hbm.at[pl.ds(base + off, _CHUNK)])
            return carry

        lax.fori_loop(0, nch, body, 0)

    return gather_k


def _xpose_body(g_ref, o_ref):
    y = g_ref[0, :, 0:64]
    o_ref[0] = y.T


def _transpose_out(g3, L, B, H):
    return pl.pallas_call(
        _xpose_body,
        grid=(L,),
        in_specs=[pl.BlockSpec((1, B, 2 * H), lambda i: (i, 0, 0))],
        out_specs=pl.BlockSpec((1, H, B), lambda i: (i, 0, 0)),
        out_shape=jax.ShapeDtypeStruct((L, H, B), jnp.float32),
    )(g3)


def kernel(seq, table, W, b):
    B, L = seq.shape
    V, H = table.shape
    t2 = _transform_table(table.T, W, b.reshape(1, H))
    idx = seq.T.reshape(B * L)
    g = _make_gather(B * L, 2 * H)(t2, idx)
    g3 = g.reshape(L, B, 2 * H)
    out_t = _transpose_out(g3, L, B, H)
    return out_t.transpose(2, 0, 1)
